# Initial kernel scaffold; baseline (speedup 1.0000x reference)
import jax
import jax.numpy as jnp
from jax.experimental import pallas as pl


def kernel(x, w_wide, b_wide, V, w1, b1, w2, b2, w3, b3, w_out, b_out):
    raise NotImplementedError("write your pallas kernel here")


# speedup vs baseline: 1.0692x; 1.0692x over previous
import jax
import jax.numpy as jnp
from jax.experimental import pallas as pl
from jax.experimental.pallas import tpu as pltpu

_BM = 1024


def _nfm_body(x_ref, wlin_ref, V_ref, w1_ref, b1_ref, w2_ref, b2_ref,
              w3_ref, b3_ref, wout_ref, bias_ref, o_ref):
    x = x_ref[...]
    V = V_ref[...]
    xv = jnp.dot(x, V, preferred_element_type=jnp.float32)
    x2v2 = jnp.dot(x * x, V * V, preferred_element_type=jnp.float32)
    t = 0.5 * (xv * xv + x2v2)
    t = jnp.maximum(jnp.dot(t, w1_ref[...], preferred_element_type=jnp.float32) + b1_ref[...], 0.0)
    t = jnp.maximum(jnp.dot(t, w2_ref[...], preferred_element_type=jnp.float32) + b2_ref[...], 0.0)
    t = jnp.maximum(jnp.dot(t, w3_ref[...], preferred_element_type=jnp.float32) + b3_ref[...], 0.0)
    z = (jnp.dot(x, wlin_ref[...], preferred_element_type=jnp.float32)
         + jnp.dot(t, wout_ref[...], preferred_element_type=jnp.float32)
         + bias_ref[...])
    o_ref[...] = jax.nn.sigmoid(z)


def kernel(x, w_wide, b_wide, V, w1, b1, w2, b2, w3, b3, w_out, b_out):
    B, F = x.shape
    K = V.shape[1]
    H1 = w1.shape[1]
    H2 = w2.shape[1]
    H3 = w3.shape[1]

    H2p = 128
    w2p = jnp.zeros((H1, H2p), jnp.float32).at[:, :H2].set(w2)
    b2p = jnp.zeros((1, H2p), jnp.float32).at[:, :H2].set(b2)
    w3p = jnp.zeros((H2p, H3), jnp.float32).at[:H2, :].set(w3)

    b1r = b1.reshape(1, H1)
    b3r = b3.reshape(1, H3)
    bias = (b_wide + b_out).reshape(1, 1)

    grid = (B // _BM,)
    full = lambda shape: pl.BlockSpec(shape, lambda i: (0,) * len(shape))
    out = pl.pallas_call(
        _nfm_body,
        out_shape=jax.ShapeDtypeStruct((B, 1), jnp.float32),
        grid=grid,
        in_specs=[
            pl.BlockSpec((_BM, F), lambda i: (i, 0)),
            full((F, 1)),
            full((F, K)),
            full((K, H1)),
            full((1, H1)),
            full((H1, H2p)),
            full((1, H2p)),
            full((H2p, H3)),
            full((1, H3)),
            full((H3, 1)),
            full((1, 1)),
        ],
        out_specs=pl.BlockSpec((_BM, 1), lambda i: (i, 0)),
        compiler_params=pltpu.CompilerParams(
            dimension_semantics=("parallel",),
        ),
        name="nfm_fused",
    )(x, w_wide, V, w1, b1r, w2p, b2p, w3p, b3r, w_out, bias)
    return out


# speedup vs baseline: 1.2594x; 1.1778x over previous
import jax
import jax.numpy as jnp
from jax.experimental import pallas as pl
from jax.experimental.pallas import tpu as pltpu

_BM = 2048


def _nfm_body(x_ref, wlin_ref, Vs_ref, w1_ref, b1_ref, w2_ref, b2_ref,
              w3_ref, b3_ref, wout_ref, bias_ref, o_ref):
    x = x_ref[...]
    Vs = Vs_ref[...]
    xv = jnp.dot(x, Vs, preferred_element_type=jnp.float32)
    x2v2 = jnp.dot(x * x, Vs * Vs, preferred_element_type=jnp.float32)
    t = xv * xv + x2v2
    t = jnp.maximum(jnp.dot(t, w1_ref[...], preferred_element_type=jnp.float32) + b1_ref[...], 0.0)
    t = jnp.maximum(jnp.dot(t, w2_ref[...], preferred_element_type=jnp.float32) + b2_ref[...], 0.0)
    t = jnp.maximum(jnp.dot(t, w3_ref[...], preferred_element_type=jnp.float32) + b3_ref[...], 0.0)
    z = (jnp.dot(x, wlin_ref[...], preferred_element_type=jnp.float32)
         + jnp.dot(t, wout_ref[...], preferred_element_type=jnp.float32)
         + bias_ref[...])
    o_ref[...] = jax.nn.sigmoid(z)


def kernel(x, w_wide, b_wide, V, w1, b1, w2, b2, w3, b3, w_out, b_out):
    B, F = x.shape
    K = V.shape[1]
    H1 = w1.shape[1]
    H2 = w2.shape[1]
    H3 = w3.shape[1]

    Vs = V * jnp.float32(0.7071067811865476)
    b1r = b1.reshape(1, H1)
    b2r = b2.reshape(1, H2)
    b3r = b3.reshape(1, H3)
    bias = (b_wide + b_out).reshape(1, 1)

    grid = (B // _BM,)
    full = lambda shape: pl.BlockSpec(shape, lambda i: (0,) * len(shape))
    out = pl.pallas_call(
        _nfm_body,
        out_shape=jax.ShapeDtypeStruct((B, 1), jnp.float32),
        grid=grid,
        in_specs=[
            pl.BlockSpec((_BM, F), lambda i: (i, 0)),
            full((F, 1)),
            full((F, K)),
            full((K, H1)),
            full((1, H1)),
            full((H1, H2)),
            full((1, H2)),
            full((H2, H3)),
            full((1, H3)),
            full((H3, 1)),
            full((1, 1)),
        ],
        out_specs=pl.BlockSpec((_BM, 1), lambda i: (i, 0)),
        compiler_params=pltpu.CompilerParams(
            dimension_semantics=("parallel",),
        ),
        name="nfm_fused",
    )(x, w_wide, Vs, w1, b1r, w2, b2r, w3, b3r, w_out, bias)
    return out


# speedup vs baseline: 1.2676x; 1.0066x over previous
import jax
import jax.numpy as jnp
from jax.experimental import pallas as pl
from jax.experimental.pallas import tpu as pltpu

_BM = 2048


def _nfm_body(x_ref, wlin_ref, V_ref, w1_ref, b1_ref, w2_ref, b2_ref,
              w3_ref, b3_ref, wout_ref, bias_ref, o_ref):
    x = x_ref[...]
    V = V_ref[...]
    xv = jnp.dot(x, V, preferred_element_type=jnp.float32)
    x2v2 = jnp.dot(x * x, V * V, preferred_element_type=jnp.float32)
    t = 0.5 * (xv * xv + x2v2)
    t = jnp.maximum(jnp.dot(t, w1_ref[...], preferred_element_type=jnp.float32) + b1_ref[...], 0.0)
    t = jnp.maximum(jnp.dot(t, w2_ref[...], preferred_element_type=jnp.float32) + b2_ref[...], 0.0)
    t = jnp.maximum(jnp.dot(t, w3_ref[...], preferred_element_type=jnp.float32) + b3_ref[...], 0.0)
    z = (jnp.dot(x, wlin_ref[...], preferred_element_type=jnp.float32)
         + jnp.dot(t, wout_ref[...], preferred_element_type=jnp.float32)
         + bias_ref[...])
    o_ref[...] = jax.nn.sigmoid(z)


def kernel(x, w_wide, b_wide, V, w1, b1, w2, b2, w3, b3, w_out, b_out):
    B, F = x.shape
    K = V.shape[1]
    H1 = w1.shape[1]
    H2 = w2.shape[1]
    H3 = w3.shape[1]

    b1r = b1.reshape(1, H1)
    b2r = b2.reshape(1, H2)
    b3r = b3.reshape(1, H3)
    bias = (b_wide + b_out).reshape(1, 1)

    grid = (B // _BM,)
    full = lambda shape: pl.BlockSpec(shape, lambda i: (0,) * len(shape))
    out = pl.pallas_call(
        _nfm_body,
        out_shape=jax.ShapeDtypeStruct((B, 1), jnp.float32),
        grid=grid,
        in_specs=[
            pl.BlockSpec((_BM, F), lambda i: (i, 0)),
            full((F, 1)),
            full((F, K)),
            full((K, H1)),
            full((1, H1)),
            full((H1, H2)),
            full((1, H2)),
            full((H2, H3)),
            full((1, H3)),
            full((H3, 1)),
            full((1, 1)),
        ],
        out_specs=pl.BlockSpec((_BM, 1), lambda i: (i, 0)),
        compiler_params=pltpu.CompilerParams(
            dimension_semantics=("parallel",),
        ),
        name="nfm_fused",
    )(x, w_wide, V, w1, b1r, w2, b2r, w3, b3r, w_out, bias)
    return out


# speedup vs baseline: 1.2681x; 1.0004x over previous
import jax
import jax.numpy as jnp
from jax.experimental import pallas as pl
from jax.experimental.pallas import tpu as pltpu

_BM = 2048


def _nfm_body(x_ref, wlin_ref, bw_ref, V_ref, w1_ref, b1_ref, w2_ref, b2_ref,
              w3_ref, b3_ref, wout_ref, bo_ref, o_ref):
    x = x_ref[...]
    V = V_ref[...]
    xv = jnp.dot(x, V, preferred_element_type=jnp.float32)
    x2v2 = jnp.dot(x * x, V * V, preferred_element_type=jnp.float32)
    t = 0.5 * (xv * xv + x2v2)
    t = jnp.maximum(jnp.dot(t, w1_ref[...], preferred_element_type=jnp.float32) + b1_ref[...].reshape(1, -1), 0.0)
    t = jnp.maximum(jnp.dot(t, w2_ref[...], preferred_element_type=jnp.float32) + b2_ref[...].reshape(1, -1), 0.0)
    t = jnp.maximum(jnp.dot(t, w3_ref[...], preferred_element_type=jnp.float32) + b3_ref[...].reshape(1, -1), 0.0)
    z = (jnp.dot(x, wlin_ref[...], preferred_element_type=jnp.float32)
         + jnp.dot(t, wout_ref[...], preferred_element_type=jnp.float32)
         + (bw_ref[...] + bo_ref[...]).reshape(1, 1))
    o_ref[...] = jax.nn.sigmoid(z)


def kernel(x, w_wide, b_wide, V, w1, b1, w2, b2, w3, b3, w_out, b_out):
    B, F = x.shape
    K = V.shape[1]
    H1 = w1.shape[1]
    H2 = w2.shape[1]
    H3 = w3.shape[1]

    grid = (B // _BM,)
    full = lambda shape: pl.BlockSpec(shape, lambda i: (0,) * len(shape))
    out = pl.pallas_call(
        _nfm_body,
        out_shape=jax.ShapeDtypeStruct((B, 1), jnp.float32),
        grid=grid,
        in_specs=[
            pl.BlockSpec((_BM, F), lambda i: (i, 0)),
            full((F, 1)),
            full((1,)),
            full((F, K)),
            full((K, H1)),
            full((H1,)),
            full((H1, H2)),
            full((H2,)),
            full((H2, H3)),
            full((H3,)),
            full((H3, 1)),
            full((1,)),
        ],
        out_specs=pl.BlockSpec((_BM, 1), lambda i: (i, 0)),
        compiler_params=pltpu.CompilerParams(
            dimension_semantics=("parallel",),
        ),
        name="nfm_fused",
    )(x, w_wide, b_wide, V, w1, b1, w2, b2, w3, b3, w_out, b_out)
    return out


# speedup vs baseline: 1.6774x; 1.3227x over previous
import jax
import jax.numpy as jnp
from jax.experimental import pallas as pl
from jax.experimental.pallas import tpu as pltpu

_BM = 8192


_CHUNK = 2048


def _nfm_body(x_ref, wlin_ref, bw_ref, V_ref, w1_ref, b1_ref, w2_ref, b2_ref,
              w3_ref, b3_ref, wout_ref, bo_ref, o_ref):
    V = V_ref[...]
    V2 = V * V
    w1 = w1_ref[...]
    w2 = w2_ref[...]
    w3 = w3_ref[...]
    wlin = wlin_ref[...]
    wout = wout_ref[...]
    b1 = b1_ref[...].reshape(1, -1)
    b2 = b2_ref[...].reshape(1, -1)
    b3 = b3_ref[...].reshape(1, -1)
    bias = (bw_ref[...] + bo_ref[...]).reshape(1, 1)
    for c in range(_BM // _CHUNK):
        rows = pl.ds(c * _CHUNK, _CHUNK)
        x = x_ref[rows, :]
        xv = jnp.dot(x, V, preferred_element_type=jnp.float32)
        x2v2 = jnp.dot(x * x, V2, preferred_element_type=jnp.float32)
        t = 0.5 * (xv * xv + x2v2)
        t = jnp.maximum(jnp.dot(t, w1, preferred_element_type=jnp.float32) + b1, 0.0)
        t = jnp.maximum(jnp.dot(t, w2, preferred_element_type=jnp.float32) + b2, 0.0)
        t = jnp.maximum(jnp.dot(t, w3, preferred_element_type=jnp.float32) + b3, 0.0)
        z = (jnp.dot(x, wlin, preferred_element_type=jnp.float32)
             + jnp.dot(t, wout, preferred_element_type=jnp.float32)
             + bias)
        o_ref[rows, :] = jax.nn.sigmoid(z)


def kernel(x, w_wide, b_wide, V, w1, b1, w2, b2, w3, b3, w_out, b_out):
    B, F = x.shape
    K = V.shape[1]
    H1 = w1.shape[1]
    H2 = w2.shape[1]
    H3 = w3.shape[1]

    grid = (B // _BM,)
    full = lambda shape: pl.BlockSpec(shape, lambda i: (0,) * len(shape))
    out = pl.pallas_call(
        _nfm_body,
        out_shape=jax.ShapeDtypeStruct((B, 1), jnp.float32),
        grid=grid,
        in_specs=[
            pl.BlockSpec((_BM, F), lambda i: (i, 0)),
            full((F, 1)),
            full((1,)),
            full((F, K)),
            full((K, H1)),
            full((H1,)),
            full((H1, H2)),
            full((H2,)),
            full((H2, H3)),
            full((H3,)),
            full((H3, 1)),
            full((1,)),
        ],
        out_specs=pl.BlockSpec((_BM, 1), lambda i: (i, 0)),
        compiler_params=pltpu.CompilerParams(
            dimension_semantics=("parallel",),
            vmem_limit_bytes=56 * 1024 * 1024,
        ),
        name="nfm_fused",
    )(x, w_wide, b_wide, V, w1, b1, w2, b2, w3, b3, w_out, b_out)
    return out


# speedup vs baseline: 1.8304x; 1.0912x over previous
import jax
import jax.numpy as jnp
from jax.experimental import pallas as pl
from jax.experimental.pallas import tpu as pltpu

_BM = 8192
_CHUNK = 2048

_DN_RHS_T = (((1,), (1,)), ((), ()))


def _nfm_body(x_ref, wlin_ref, bw_ref, V_ref, w1_ref, b1_ref, w2_ref, b2_ref,
              w3_ref, b3_ref, wout_ref, bo_ref, o_ref):
    V = V_ref[...]
    V2 = V * V
    w1 = w1_ref[...]
    w2 = w2_ref[...]
    w3 = w3_ref[...]
    wlin_t = wlin_ref[...]
    wout_t = wout_ref[...]
    b1 = b1_ref[...].reshape(1, -1)
    b2 = b2_ref[...].reshape(1, -1)
    b3 = b3_ref[...].reshape(1, -1)
    bias = (bw_ref[...] + bo_ref[...]).reshape(1, 1)
    for c in range(_BM // _CHUNK):
        rows = pl.ds(c * _CHUNK, _CHUNK)
        x = x_ref[rows, :]
        xv = jnp.dot(x, V, preferred_element_type=jnp.float32)
        x2v2 = jnp.dot(x * x, V2, preferred_element_type=jnp.float32)
        t = 0.5 * (xv * xv + x2v2)
        t = jnp.maximum(jnp.dot(t, w1, preferred_element_type=jnp.float32) + b1, 0.0)
        t = jnp.maximum(jnp.dot(t, w2, preferred_element_type=jnp.float32) + b2, 0.0)
        t = jnp.maximum(jnp.dot(t, w3, preferred_element_type=jnp.float32) + b3, 0.0)
        z = (jax.lax.dot_general(wlin_t, x, _DN_RHS_T, preferred_element_type=jnp.float32)
             + jax.lax.dot_general(wout_t, t, _DN_RHS_T, preferred_element_type=jnp.float32)
             + bias)
        o_ref[:, pl.ds(c * _CHUNK, _CHUNK)] = jax.nn.sigmoid(z)


def kernel(x, w_wide, b_wide, V, w1, b1, w2, b2, w3, b3, w_out, b_out):
    B, F = x.shape
    K = V.shape[1]
    H1 = w1.shape[1]
    H2 = w2.shape[1]
    H3 = w3.shape[1]

    wlin_t = w_wide.reshape(1, F)
    wout_t = w_out.reshape(1, H3)

    grid = (B // _BM,)
    full = lambda shape: pl.BlockSpec(shape, lambda i: (0,) * len(shape))
    out = pl.pallas_call(
        _nfm_body,
        out_shape=jax.ShapeDtypeStruct((1, B), jnp.float32),
        grid=grid,
        in_specs=[
            pl.BlockSpec((_BM, F), lambda i: (i, 0)),
            full((1, F)),
            full((1,)),
            full((F, K)),
            full((K, H1)),
            full((H1,)),
            full((H1, H2)),
            full((H2,)),
            full((H2, H3)),
            full((H3,)),
            full((1, H3)),
            full((1,)),
        ],
        out_specs=pl.BlockSpec((1, _BM), lambda i: (0, i)),
        compiler_params=pltpu.CompilerParams(
            dimension_semantics=("parallel",),
        ),
        name="nfm_fused",
    )(x, wlin_t, b_wide, V, w1, b1, w2, b2, w3, b3, wout_t, b_out)
    return out.reshape(B, 1)


# speedup vs baseline: 2.0100x; 1.0981x over previous
import jax
import jax.numpy as jnp
from jax.experimental import pallas as pl
from jax.experimental.pallas import tpu as pltpu

_BM = 8192
_CHUNK = 4096

_DN_RHS_T = (((1,), (1,)), ((), ()))


def _nfm_body(x_ref, wlin_ref, bw_ref, V_ref, w1_ref, b1_ref, w2_ref, b2_ref,
              w3_ref, b3_ref, wout_ref, bo_ref, o_ref):
    V = V_ref[...]
    V2 = V * V
    w1 = w1_ref[...]
    w2 = w2_ref[...]
    w3 = w3_ref[...]
    wlin_t = wlin_ref[...]
    wout_t = wout_ref[...]
    b1 = b1_ref[...].reshape(1, -1)
    b2 = b2_ref[...].reshape(1, -1)
    b3 = b3_ref[...].reshape(1, -1)
    bias = (bw_ref[...] + bo_ref[...]).reshape(1, 1)
    for c in range(_BM // _CHUNK):
        rows = pl.ds(c * _CHUNK, _CHUNK)
        x = x_ref[rows, :]
        xv = jnp.dot(x, V, preferred_element_type=jnp.float32)
        x2v2 = jnp.dot(x * x, V2, preferred_element_type=jnp.float32)
        t = 0.5 * (xv * xv + x2v2)
        t = jnp.maximum(jnp.dot(t, w1, preferred_element_type=jnp.float32) + b1, 0.0)
        t = jnp.maximum(jnp.dot(t, w2, preferred_element_type=jnp.float32) + b2, 0.0)
        t = jnp.maximum(jnp.dot(t, w3, preferred_element_type=jnp.float32) + b3, 0.0)
        z = (jax.lax.dot_general(wlin_t, x, _DN_RHS_T, preferred_element_type=jnp.float32)
             + jax.lax.dot_general(wout_t, t, _DN_RHS_T, preferred_element_type=jnp.float32)
             + bias)
        o_ref[:, pl.ds(c * _CHUNK, _CHUNK)] = jax.nn.sigmoid(z)


def kernel(x, w_wide, b_wide, V, w1, b1, w2, b2, w3, b3, w_out, b_out):
    B, F = x.shape
    K = V.shape[1]
    H1 = w1.shape[1]
    H2 = w2.shape[1]
    H3 = w3.shape[1]

    wlin_t = w_wide.reshape(1, F)
    wout_t = w_out.reshape(1, H3)

    grid = (B // _BM,)
    full = lambda shape: pl.BlockSpec(shape, lambda i: (0,) * len(shape))
    out = pl.pallas_call(
        _nfm_body,
        out_shape=jax.ShapeDtypeStruct((1, B), jnp.float32),
        grid=grid,
        in_specs=[
            pl.BlockSpec((_BM, F), lambda i: (i, 0)),
            full((1, F)),
            full((1,)),
            full((F, K)),
            full((K, H1)),
            full((H1,)),
            full((H1, H2)),
            full((H2,)),
            full((H2, H3)),
            full((H3,)),
            full((1, H3)),
            full((1,)),
        ],
        out_specs=pl.BlockSpec((1, _BM), lambda i: (0, i)),
        compiler_params=pltpu.CompilerParams(
            dimension_semantics=("parallel",),
        ),
        name="nfm_fused",
    )(x, wlin_t, b_wide, V, w1, b1, w2, b2, w3, b3, wout_t, b_out)
    return out.reshape(B, 1)
